# baseline (speedup 1.0000x reference)
import jax
import jax.numpy as jnp
from jax import lax
from jax.experimental import pallas as pl
from jax.experimental.pallas import tpu as pltpu
from jax.experimental.pallas import tpu_sc as plsc

_DIM = 32
_LANES = 16
_NUM_CORES = 2
_NUM_SUBCORES = 16
_NUM_WORKERS = _NUM_CORES * _NUM_SUBCORES
_ROWS_PER_STREAM = 128
_BITREV = (0, 8, 4, 12, 2, 10, 6, 14, 1, 9, 5, 13, 3, 11, 7, 15)


def _transe_body(hidx_hbm, ridx_hbm, tidx_hbm, ent_hbm, rel_hbm, out_hbm,
                 hidx_v, ridx_v, tidx_v, h_v, r_v, t_v, out_v, sem):
    wid = lax.axis_index("s") * _NUM_CORES + lax.axis_index("c")
    nchunks = hidx_v.shape[0]
    n = nchunks * _ROWS_PER_STREAM

    pltpu.sync_copy(hidx_hbm.at[wid], hidx_v)
    pltpu.sync_copy(ridx_hbm.at[wid], ridx_v)
    pltpu.sync_copy(tidx_hbm.at[wid], tidx_v)

    descs = []
    for c in range(nchunks):
        dst = pl.ds(c * _ROWS_PER_STREAM, _ROWS_PER_STREAM)
        descs.append(pltpu.async_copy(ent_hbm.at[hidx_v.at[c]], h_v.at[dst], sem))
        descs.append(pltpu.async_copy(rel_hbm.at[ridx_v.at[c]], r_v.at[dst], sem))
        descs.append(pltpu.async_copy(ent_hbm.at[tidx_v.at[c]], t_v.at[dst], sem))
    for dsc in descs:
        dsc.wait()

    lanes = lax.iota(jnp.int32, _LANES)
    lo = pl.ds(0, _LANES)
    hi = pl.ds(_LANES, _LANES)

    def merge(a, b, k):
        fa = a + a[lanes ^ k]
        fb = b + b[lanes ^ k]
        return jnp.where((lanes & k) == 0, fa, fb)

    def group_body(g, carry):
        base = g * _LANES
        s = []
        for j in _BITREV:
            row = base + j
            a0 = jnp.abs(h_v[row, lo] + r_v[row, lo] - t_v[row, lo])
            a1 = jnp.abs(h_v[row, hi] + r_v[row, hi] - t_v[row, hi])
            s.append(a0 + a1)
        for k in (8, 4, 2, 1):
            s = [merge(s[2 * i], s[2 * i + 1], k) for i in range(len(s) // 2)]
        out_v[pl.ds(base, _LANES)] = s[0]
        return carry

    lax.fori_loop(0, n // _LANES, group_body, 0)

    pltpu.sync_copy(out_v, out_hbm.at[pl.ds(wid * n, n)])


def kernel(pos_triples, neg_triples, entity_emb, relation_emb):
    trip = jnp.concatenate([pos_triples, neg_triples], axis=0)
    total = trip.shape[0]
    n = total // _NUM_WORKERS
    nchunks = n // _ROWS_PER_STREAM
    hidx = trip[:, 0].reshape(_NUM_WORKERS, nchunks, _ROWS_PER_STREAM)
    ridx = trip[:, 1].reshape(_NUM_WORKERS, nchunks, _ROWS_PER_STREAM)
    tidx = trip[:, 2].reshape(_NUM_WORKERS, nchunks, _ROWS_PER_STREAM)

    mesh = plsc.VectorSubcoreMesh(core_axis_name="c", subcore_axis_name="s")
    f = pl.kernel(
        _transe_body,
        mesh=mesh,
        compiler_params=pltpu.CompilerParams(use_tc_tiling_on_sc=False),
        out_type=jax.ShapeDtypeStruct((total,), jnp.float32),
        scratch_types=[
            pltpu.VMEM((nchunks, _ROWS_PER_STREAM), jnp.int32),
            pltpu.VMEM((nchunks, _ROWS_PER_STREAM), jnp.int32),
            pltpu.VMEM((nchunks, _ROWS_PER_STREAM), jnp.int32),
            pltpu.VMEM((n, _DIM), jnp.float32),
            pltpu.VMEM((n, _DIM), jnp.float32),
            pltpu.VMEM((n, _DIM), jnp.float32),
            pltpu.VMEM((n,), jnp.float32),
            pltpu.SemaphoreType.DMA,
        ],
    )
    return f(hidx, ridx, tidx, entity_emb, relation_emb)
